# baseline (speedup 1.0000x reference)
import functools

import jax
import jax.numpy as jnp
from jax.experimental import pallas as pl

N = 10000
D = 128
H = 8
OC = 8
F = 64
TW = 80


def _prep_body(x_ref, w_ref, as_ref, ad_ref, src_ref, dst_ref):
    h = jnp.dot(x_ref[...], w_ref[...], preferred_element_type=jnp.float32)
    a_s = jnp.dot(h, as_ref[...], preferred_element_type=jnp.float32)
    a_d = jnp.dot(h, ad_ref[...], preferred_element_type=jnp.float32)
    z = jnp.max(a_s) + jnp.max(a_d)
    c = jnp.where(z > 0.0, z, 0.2 * z)
    src_ref[:, :F] = h
    src_ref[:, F:F + H] = a_s
    src_ref[:, F + H:] = jnp.zeros((N, TW - F - H), jnp.float32)
    dst_ref[:, :H] = a_d - c
    dst_ref[:, H:] = a_d - 5.0 * c


def _prep(x, w, a_s, a_d):
    return pl.pallas_call(
        _prep_body,
        out_shape=(
            jax.ShapeDtypeStruct((N, TW), jnp.float32),
            jax.ShapeDtypeStruct((N, 2 * H), jnp.float32),
        ),
    )(x, w, a_s, a_d)


def _combine_body(mode, acc_ref, src_ref, dst_ref, r_ref, b_ref, out_ref):
    num = acc_ref[0, :, :F] + acc_ref[1, :, :F]
    den = acc_ref[0, :, F:F + H] + acc_ref[1, :, F:F + H]
    h = src_ref[:, :F]
    a_s = src_ref[:, F:F + H]
    p = dst_ref[:, :H]
    q = dst_ref[:, H:]
    wself = jnp.exp(jnp.maximum(a_s + p, 0.2 * (a_s + q)))
    r = r_ref[...]
    den = jnp.dot(den + wself, r, preferred_element_type=jnp.float32)
    num = num + h * jnp.dot(wself, r, preferred_element_type=jnp.float32)
    out = num / (den + 1e-16) + b_ref[...]
    if mode == "elu":
        out = jnp.where(out > 0.0, out, jnp.exp(jnp.minimum(out, 0.0)) - 1.0)
    else:
        m = jnp.max(out, axis=1, keepdims=True)
        lse = jnp.log(jnp.sum(jnp.exp(out - m), axis=1, keepdims=True)) + m
        out = out - lse
    out_ref[...] = out


def _combine(mode, acc, src_tab, dst_tab, r, b):
    return pl.pallas_call(
        functools.partial(_combine_body, mode),
        out_shape=jax.ShapeDtypeStruct((N, F), jnp.float32),
    )(acc, src_tab, dst_tab, r, b)


def _edge_pass(src, dst, src_tab, dst_tab):
    a_se = src_tab[src, F:F + H]
    pe = dst_tab[dst, :H]
    qe = dst_tab[dst, H:]
    w = jnp.exp(jnp.maximum(a_se + pe, 0.2 * (a_se + qe)))
    wx = jnp.repeat(w, OC, axis=1)
    num = jax.ops.segment_sum(src_tab[src, :F] * wx, dst, num_segments=N)
    den = jax.ops.segment_sum(w, dst, num_segments=N)
    acc = jnp.zeros((2, N, TW), jnp.float32)
    acc = acc.at[0, :, :F].set(num)
    acc = acc.at[0, :, F:F + H].set(den)
    return acc


def kernel(x, edge_index, W1, a_src1, a_dst1, b1, W2, a_src2, a_dst2, b2):
    src = edge_index[0]
    dst = edge_index[1]

    eye = jnp.eye(H, dtype=jnp.float32)
    as1 = (a_src1[0][:, :, None] * eye[:, None, :]).reshape(F, H)
    ad1 = (a_dst1[0][:, :, None] * eye[:, None, :]).reshape(F, H)
    as2 = jnp.broadcast_to(a_src2[0, 0][:, None], (F, H)).astype(jnp.float32)
    ad2 = jnp.broadcast_to(a_dst2[0, 0][:, None], (F, H)).astype(jnp.float32)
    r = jnp.repeat(jnp.eye(H, dtype=jnp.float32), OC, axis=1)

    src_tab1, dst_tab1 = _prep(x, W1, as1, ad1)
    acc1 = _edge_pass(src, dst, src_tab1, dst_tab1)
    x2 = _combine("elu", acc1, src_tab1, dst_tab1, r, b1)

    src_tab2, dst_tab2 = _prep(x2, W2, as2, ad2)
    acc2 = _edge_pass(src, dst, src_tab2, dst_tab2)
    return _combine("logsm", acc2, src_tab2, dst_tab2, r, b2)


# speedup vs baseline: 346.7812x; 346.7812x over previous
import functools

import jax
import jax.numpy as jnp
from jax import lax
from jax.experimental import pallas as pl
from jax.experimental.pallas import tpu as pltpu
from jax.experimental.pallas import tpu_sc as plsc

N = 10000
D = 128
H = 8
OC = 8
F = 64
TW = 128


def _prep_body(x_ref, w_ref, as_ref, ad_ref, tab_ref):
    h = jnp.dot(x_ref[...], w_ref[...], preferred_element_type=jnp.float32)
    a_s = jnp.dot(h, as_ref[...], preferred_element_type=jnp.float32)
    a_d = jnp.dot(h, ad_ref[...], preferred_element_type=jnp.float32)
    z = jnp.max(a_s) + jnp.max(a_d)
    c = jnp.where(z > 0.0, z, 0.2 * z)
    tab_ref[:, :F] = h
    tab_ref[:, F:F + H] = a_s
    tab_ref[:, F + H:F + 2 * H] = jnp.zeros((N, H), jnp.float32)
    tab_ref[:, 80:88] = a_d - c
    tab_ref[:, 88:96] = a_d - 5.0 * c
    tab_ref[:, 96:] = jnp.zeros((N, TW - 96), jnp.float32)


def _prep(x, w, a_s, a_d):
    return pl.pallas_call(
        _prep_body,
        out_shape=jax.ShapeDtypeStruct((N, TW), jnp.float32),
    )(x, w, a_s, a_d)


def _combine_body(mode, acc_ref, tab_ref, r_ref, b_ref, out_ref):
    num = acc_ref[0, :N, :F] + acc_ref[1, :N, :F]
    den = acc_ref[0, :N, F:F + H] + acc_ref[1, :N, F:F + H]
    h = tab_ref[:, :F]
    a_s = tab_ref[:, F:F + H]
    p = tab_ref[:, 80:88]
    q = tab_ref[:, 88:96]
    wself = jnp.exp(jnp.maximum(a_s + p, 0.2 * (a_s + q)))
    r = r_ref[...]
    den = jnp.dot(den + wself, r, preferred_element_type=jnp.float32)
    num = num + h * jnp.dot(wself, r, preferred_element_type=jnp.float32)
    out = num / (den + 1e-16) + b_ref[...]
    if mode == "elu":
        out = jnp.where(out > 0.0, out, jnp.exp(jnp.minimum(out, 0.0)) - 1.0)
    else:
        m = jnp.max(out, axis=1, keepdims=True)
        lse = jnp.log(jnp.sum(jnp.exp(out - m), axis=1, keepdims=True)) + m
        out = out - lse
    out_ref[...] = out


def _combine(mode, acc, tab, r, b):
    return pl.pallas_call(
        functools.partial(_combine_body, mode),
        out_shape=jax.ShapeDtypeStruct((N, F), jnp.float32),
    )(acc, tab, r, b)


E = 320000
NWORK = 32
CH = 128
NCHUNK = E // CH
NBLK = (N + CH - 1) // CH
N_PAD = NBLK * CH


def _vgather(x, idx):
    dnums = lax.GatherDimensionNumbers(
        offset_dims=(), collapsed_slice_dims=(0,), start_index_map=(0,))
    return lax.gather(x, idx[:, None], dnums, (1,),
                      mode=lax.GatherScatterMode.PROMISE_IN_BOUNDS)


def _edge_sc_body(src_hbm, dst_hbm, tab_hbm, zrow_hbm, out_hbm,
                  sid_v, did_v, srows_v, drows_v, wbuf_v, acc_sh):
    c = lax.axis_index("c")
    s = lax.axis_index("s")
    w = s * 2 + c

    lane16 = lax.iota(jnp.int32, 16)
    sel = lane16 >> 3
    selm = sel == 1

    pltpu.sync_copy(zrow_hbm, wbuf_v)
    for j in range(5):
        b = s + 16 * j

        @pl.when(b < NBLK)
        def _():
            r = pl.multiple_of(b * CH, CH)
            pltpu.sync_copy(wbuf_v, acc_sh.at[pl.ds(r, CH)])
    plsc.subcore_barrier()

    def chunk_body(k, _):
        cid = w + NWORK * k
        base = pl.multiple_of(cid * CH, CH)
        pltpu.sync_copy(src_hbm.at[pl.ds(base, CH)], sid_v)
        pltpu.sync_copy(dst_hbm.at[pl.ds(base, CH)], did_v)
        pltpu.sync_copy(tab_hbm.at[sid_v], srows_v)
        pltpu.sync_copy(tab_hbm.at[did_v], drows_v)

        def pair_body(i, _):
            e0 = 2 * i
            e1 = 2 * i + 1
            as_pair = jnp.where(selm, srows_v[e1, pl.ds(F - H, 16)],
                                srows_v[e0, pl.ds(F, 16)])
            p_pair = jnp.where(selm, drows_v[e1, pl.ds(72, 16)],
                               drows_v[e0, pl.ds(80, 16)])
            q_pair = jnp.where(selm, drows_v[e1, pl.ds(80, 16)],
                               drows_v[e0, pl.ds(88, 16)])
            wv = jnp.exp(jnp.maximum(as_pair + p_pair,
                                     0.2 * (as_pair + q_pair)))
            wbuf_v[e0, pl.ds(F, 16)] = wv
            wbuf_v[e1, pl.ds(F - H, 16)] = wv
            for v in range(4):
                perm0 = sel + 2 * v
                wx0 = _vgather(wv, perm0)
                wx1 = _vgather(wv, perm0 + 8)
                h0 = srows_v[e0, pl.ds(16 * v, 16)]
                h1 = srows_v[e1, pl.ds(16 * v, 16)]
                wbuf_v[e0, pl.ds(16 * v, 16)] = h0 * wx0
                wbuf_v[e1, pl.ds(16 * v, 16)] = h1 * wx1
            return 0

        lax.fori_loop(0, CH // 2, pair_body, 0)
        pltpu.sync_copy(wbuf_v, acc_sh.at[did_v], add=True)
        return 0

    nk = jnp.where(w < NCHUNK - (NCHUNK // NWORK) * NWORK,
                   NCHUNK // NWORK + 1, NCHUNK // NWORK)
    lax.fori_loop(0, nk, chunk_body, 0)
    plsc.subcore_barrier()

    for j in range(5):
        b = s + 16 * j

        @pl.when(b < NBLK)
        def _():
            r = pl.multiple_of(b * CH, CH)
            pltpu.sync_copy(acc_sh.at[pl.ds(r, CH)], srows_v)
            pltpu.sync_copy(srows_v, out_hbm.at[c, pl.ds(r, CH)])


def _edge_sc(src, dst, tab, zrow):
    mesh = plsc.VectorSubcoreMesh(core_axis_name="c", subcore_axis_name="s")
    f = pl.kernel(
        _edge_sc_body,
        out_type=jax.ShapeDtypeStruct((2, N_PAD, TW), jnp.float32),
        mesh=mesh,
        scratch_types=[
            pltpu.VMEM((CH,), jnp.int32),
            pltpu.VMEM((CH,), jnp.int32),
            pltpu.VMEM((CH, TW), jnp.float32),
            pltpu.VMEM((CH, TW), jnp.float32),
            pltpu.VMEM((CH, TW), jnp.float32),
            pltpu.VMEM_SHARED((N_PAD, TW), jnp.float32),
        ],
    )
    return f(src, dst, tab, zrow)


def _edge_pass(src, dst, src_tab, dst_tab):
    a_se = src_tab[src, F:F + H]
    pe = dst_tab[dst, :H]
    qe = dst_tab[dst, H:]
    w = jnp.exp(jnp.maximum(a_se + pe, 0.2 * (a_se + qe)))
    wx = jnp.repeat(w, OC, axis=1)
    num = jax.ops.segment_sum(src_tab[src, :F] * wx, dst, num_segments=N)
    den = jax.ops.segment_sum(w, dst, num_segments=N)
    acc = jnp.zeros((2, N, TW), jnp.float32)
    acc = acc.at[0, :, :F].set(num)
    acc = acc.at[0, :, F:F + H].set(den)
    return acc


def kernel(x, edge_index, W1, a_src1, a_dst1, b1, W2, a_src2, a_dst2, b2):
    src = edge_index[0]
    dst = edge_index[1]

    eye = jnp.eye(H, dtype=jnp.float32)
    as1 = (a_src1[0][:, :, None] * eye[:, None, :]).reshape(F, H)
    ad1 = (a_dst1[0][:, :, None] * eye[:, None, :]).reshape(F, H)
    as2 = jnp.broadcast_to(a_src2[0, 0][:, None], (F, H)).astype(jnp.float32)
    ad2 = jnp.broadcast_to(a_dst2[0, 0][:, None], (F, H)).astype(jnp.float32)
    r = jnp.repeat(jnp.eye(H, dtype=jnp.float32), OC, axis=1)

    zrow = jnp.zeros((CH, TW), jnp.float32)
    tab1 = _prep(x, W1, as1, ad1)
    acc1 = _edge_sc(src, dst, tab1, zrow)
    x2 = _combine("elu", acc1, tab1, r, b1)

    tab2 = _prep(x2, W2, as2, ad2)
    acc2 = _edge_sc(src, dst, tab2, zrow)
    return _combine("logsm", acc2, tab2, r, b2)


# speedup vs baseline: 346.7946x; 1.0000x over previous
import functools

import jax
import jax.numpy as jnp
from jax import lax
from jax.experimental import pallas as pl
from jax.experimental.pallas import tpu as pltpu
from jax.experimental.pallas import tpu_sc as plsc

N = 10000
D = 128
H = 8
OC = 8
F = 64
TW = 128

E = 320000
NWORK = 32
CH = 128
NCHUNK = E // CH
NBLK = (N + CH - 1) // CH
N_PAD = NBLK * CH


def _prep_body(x_ref, w_ref, as_ref, ad_ref, tab_ref):
    h = jnp.dot(x_ref[...], w_ref[...], preferred_element_type=jnp.float32)
    a_s = jnp.dot(h, as_ref[...], preferred_element_type=jnp.float32)
    a_d = jnp.dot(h, ad_ref[...], preferred_element_type=jnp.float32)
    z = jnp.max(a_s) + jnp.max(a_d)
    c = jnp.where(z > 0.0, z, 0.2 * z)
    tab_ref[:, :F] = h
    tab_ref[:, F:F + H] = a_s
    tab_ref[:, F + H:F + 2 * H] = jnp.zeros((N, H), jnp.float32)
    tab_ref[:, 80:88] = a_d - c
    tab_ref[:, 88:96] = a_d - 5.0 * c
    tab_ref[:, 96:] = jnp.zeros((N, TW - 96), jnp.float32)


def _prep(x, w, a_s, a_d):
    return pl.pallas_call(
        _prep_body,
        out_shape=jax.ShapeDtypeStruct((N, TW), jnp.float32),
    )(x, w, a_s, a_d)


def _combine_body(mode, acc_ref, tab_ref, r_ref, b_ref, out_ref):
    num = acc_ref[0, :N, :F] + acc_ref[1, :N, :F]
    den = acc_ref[0, :N, F:F + H] + acc_ref[1, :N, F:F + H]
    h = tab_ref[:, :F]
    a_s = tab_ref[:, F:F + H]
    p = tab_ref[:, 80:88]
    q = tab_ref[:, 88:96]
    wself = jnp.exp(jnp.maximum(a_s + p, 0.2 * (a_s + q)))
    r = r_ref[...]
    den = jnp.dot(den + wself, r, preferred_element_type=jnp.float32)
    num = num + h * jnp.dot(wself, r, preferred_element_type=jnp.float32)
    out = num / (den + 1e-16) + b_ref[...]
    if mode == "elu":
        out = jnp.where(out > 0.0, out, jnp.exp(jnp.minimum(out, 0.0)) - 1.0)
    else:
        m = jnp.max(out, axis=1, keepdims=True)
        lse = jnp.log(jnp.sum(jnp.exp(out - m), axis=1, keepdims=True)) + m
        out = out - lse
    out_ref[...] = out


def _combine(mode, acc, tab, r, b):
    return pl.pallas_call(
        functools.partial(_combine_body, mode),
        out_shape=jax.ShapeDtypeStruct((N, F), jnp.float32),
    )(acc, tab, r, b)


def _vgather(x, idx):
    dnums = lax.GatherDimensionNumbers(
        offset_dims=(), collapsed_slice_dims=(0,), start_index_map=(0,))
    return lax.gather(x, idx[:, None], dnums, (1,),
                      mode=lax.GatherScatterMode.PROMISE_IN_BOUNDS)


def _edge_sc_body(src_hbm, dst_hbm, tab_hbm, zrow_hbm, out_hbm,
                  sid_v, did_v, srows_v, drows_v, wbuf_v, acc_sh):
    c = lax.axis_index("c")
    s = lax.axis_index("s")
    w = s * 2 + c

    lane16 = lax.iota(jnp.int32, 16)
    lane8 = lane16 & 7
    sel = lane16 >> 3
    selm = sel == 1

    pltpu.sync_copy(zrow_hbm, wbuf_v)
    for j in range(5):
        b = s + 16 * j

        @pl.when(b < NBLK)
        def _():
            r = pl.multiple_of(b * CH, CH)
            pltpu.sync_copy(wbuf_v, acc_sh.at[pl.ds(r, CH)])
    plsc.subcore_barrier()

    def chunk_body(k, _):
        cid = w + NWORK * k
        base = pl.multiple_of(cid * CH, CH)
        pltpu.sync_copy(src_hbm.at[pl.ds(base, CH)], sid_v)
        pltpu.sync_copy(dst_hbm.at[pl.ds(base, CH)], did_v)
        pltpu.sync_copy(tab_hbm.at[sid_v], srows_v)
        pltpu.sync_copy(tab_hbm.at[did_v], drows_v)

        def pair_body(i, _):
            e0 = 2 * i
            e1 = 2 * i + 1
            as_pair = jnp.where(selm, srows_v[e1, pl.ds(F - H, 16)],
                                srows_v[e0, pl.ds(F, 16)])
            p_pair = jnp.where(selm, drows_v[e1, pl.ds(72, 16)],
                               drows_v[e0, pl.ds(80, 16)])
            q_pair = jnp.where(selm, drows_v[e1, pl.ds(80, 16)],
                               drows_v[e0, pl.ds(88, 16)])
            wv = jnp.exp(jnp.maximum(as_pair + p_pair,
                                     0.2 * (as_pair + q_pair)))
            wbuf_v[e0, pl.ds(F, 16)] = wv
            wbuf_v[e1, pl.ds(F - H, 16)] = wv
            for v in range(4):
                perm0 = sel + 2 * v
                wx0 = _vgather(wv, perm0)
                wx1 = _vgather(wv, perm0 + 8)
                h0 = srows_v[e0, pl.ds(16 * v, 16)]
                h1 = srows_v[e1, pl.ds(16 * v, 16)]
                wbuf_v[e0, pl.ds(16 * v, 16)] = h0 * wx0
                wbuf_v[e1, pl.ds(16 * v, 16)] = h1 * wx1
            return 0

        lax.fori_loop(0, CH // 2, pair_body, 0)
        pltpu.sync_copy(wbuf_v, acc_sh.at[did_v], add=True)
        return 0

    nk = jnp.where(w < NCHUNK - (NCHUNK // NWORK) * NWORK,
                   NCHUNK // NWORK + 1, NCHUNK // NWORK)
    lax.fori_loop(0, nk, chunk_body, 0)
    plsc.subcore_barrier()

    for j in range(5):
        b = s + 16 * j

        @pl.when(b < NBLK)
        def _():
            r = pl.multiple_of(b * CH, CH)
            pltpu.sync_copy(acc_sh.at[pl.ds(r, CH)], srows_v)
            pltpu.sync_copy(srows_v, out_hbm.at[c, pl.ds(r, CH)])


def _edge_sc(src, dst, tab, zrow):
    mesh = plsc.VectorSubcoreMesh(core_axis_name="c", subcore_axis_name="s")
    f = pl.kernel(
        _edge_sc_body,
        out_type=jax.ShapeDtypeStruct((2, N_PAD, TW), jnp.float32),
        mesh=mesh,
        scratch_types=[
            pltpu.VMEM((CH,), jnp.int32),
            pltpu.VMEM((CH,), jnp.int32),
            pltpu.VMEM((CH, TW), jnp.float32),
            pltpu.VMEM((CH, TW), jnp.float32),
            pltpu.VMEM((CH, TW), jnp.float32),
            pltpu.VMEM_SHARED((N_PAD, TW), jnp.float32),
        ],
    )
    return f(src, dst, tab, zrow)


def kernel(x, edge_index, W1, a_src1, a_dst1, b1, W2, a_src2, a_dst2, b2):
    src = edge_index[0]
    dst = edge_index[1]

    eye = jnp.eye(H, dtype=jnp.float32)
    as1 = (a_src1[0][:, :, None] * eye[:, None, :]).reshape(F, H)
    ad1 = (a_dst1[0][:, :, None] * eye[:, None, :]).reshape(F, H)
    as2 = jnp.broadcast_to(a_src2[0, 0][:, None], (F, H)).astype(jnp.float32)
    ad2 = jnp.broadcast_to(a_dst2[0, 0][:, None], (F, H)).astype(jnp.float32)
    r = jnp.repeat(jnp.eye(H, dtype=jnp.float32), OC, axis=1)

    zrow = jnp.zeros((CH, TW), jnp.float32)
    tab1 = _prep(x, W1, as1, ad1)
    acc1 = _edge_sc(src, dst, tab1, zrow)
    x2 = _combine("elu", acc1, tab1, r, b1)

    tab2 = _prep(x2, W2, as2, ad2)
    acc2 = _edge_sc(src, dst, tab2, zrow)
    return _combine("logsm", acc2, tab2, r, b2)
